# Initial kernel scaffold; baseline (speedup 1.0000x reference)
import jax
import jax.numpy as jnp
from jax.experimental import pallas as pl


def kernel(customer_id, article_id, product_type, colour_group, department, T_customer, T_article, T_product, T_colour, T_department, W1, b1, W2, b2):
    raise NotImplementedError("write your pallas kernel here")


# speedup vs baseline: 7.2226x; 7.2226x over previous
import functools

import jax
import jax.numpy as jnp
from jax import lax
from jax.experimental import pallas as pl
from jax.experimental.pallas import tpu as pltpu
from jax.experimental.pallas import tpu_sc as plsc

B = 16384
NC, NS = 2, 16
NW = NC * NS
BPW = B // NW
CHUNK = 128
NCHUNK = BPW // CHUNK
L = 16
NGRP = BPW // L

D_PROD, D_COL, D_DEPT = 21, 17, 24
D_SMALL = D_PROD + D_COL + D_DEPT
V_PROD, V_COL, V_DEPT = 133, 51, 301


@functools.lru_cache(maxsize=1)
def _make_sc_gather():
  mesh = plsc.VectorSubcoreMesh(core_axis_name="c", subcore_axis_name="s",
                                num_cores=NC, num_subcores=NS)

  @functools.partial(
      pl.kernel,
      out_type=(jax.ShapeDtypeStruct((B, 128), jnp.float32),
                jax.ShapeDtypeStruct((B, 128), jnp.float32),
                jax.ShapeDtypeStruct((D_SMALL, B), jnp.float32)),
      mesh=mesh,
      compiler_params=pltpu.CompilerParams(needs_layout_passes=False),
      scratch_types=(
          [pltpu.VMEM((BPW,), jnp.int32) for _ in range(5)]
          + [pltpu.VMEM((CHUNK, 128), jnp.float32) for _ in range(2)]
          + [pltpu.VMEM((V_PROD * D_PROD,), jnp.float32),
             pltpu.VMEM((V_COL * D_COL,), jnp.float32),
             pltpu.VMEM((V_DEPT * D_DEPT,), jnp.float32),
             pltpu.VMEM((D_SMALL, BPW), jnp.float32),
             pltpu.SemaphoreType.DMA]
      ),
  )
  def _sc_gather(cid, aid, pid, gid, did, tc, ta, tpf, tgf, tdf,
                 e1, e2, e3t,
                 ic, ia, ip, ig, idp, rc, ra, tpv, tgv, tdv, rst, sem):
      w = lax.axis_index("s") * NC + lax.axis_index("c")
      base = w * BPW
      pltpu.sync_copy(cid.at[pl.ds(base, BPW)], ic)
      pltpu.sync_copy(aid.at[pl.ds(base, BPW)], ia)
      pltpu.sync_copy(pid.at[pl.ds(base, BPW)], ip)
      pltpu.sync_copy(gid.at[pl.ds(base, BPW)], ig)
      pltpu.sync_copy(did.at[pl.ds(base, BPW)], idp)
      pltpu.sync_copy(tpf, tpv)
      pltpu.sync_copy(tgf, tgv)
      pltpu.sync_copy(tdf, tdv)

      def group_body(g, carry):
          r0 = g * L
          p16 = ip[pl.ds(r0, L)]
          g16 = ig[pl.ds(r0, L)]
          d16 = idp[pl.ds(r0, L)]
          pb = p16 * D_PROD
          gb = g16 * D_COL
          db = d16 * D_DEPT
          for d in range(D_PROD):
              rst[d, pl.ds(r0, L)] = plsc.load_gather(tpv, [pb + d])
          for d in range(D_COL):
              rst[D_PROD + d, pl.ds(r0, L)] = plsc.load_gather(tgv, [gb + d])
          for d in range(D_DEPT):
              rst[D_PROD + D_COL + d, pl.ds(r0, L)] = (
                  plsc.load_gather(tdv, [db + d]))
          return carry

      for c in range(NCHUNK):
          off = c * CHUNK
          d1 = pltpu.async_copy(tc.at[ic.at[pl.ds(off, CHUNK)]], rc, sem)
          d2 = pltpu.async_copy(ta.at[ia.at[pl.ds(off, CHUNK)]], ra, sem)
          if c == 0:
              lax.fori_loop(0, NGRP, group_body, 0)
          d1.wait(); d2.wait()
          pltpu.sync_copy(rc, e1.at[pl.ds(base + off, CHUNK)])
          pltpu.sync_copy(ra, e2.at[pl.ds(base + off, CHUNK)])
      pltpu.sync_copy(rst, e3t.at[pl.ds(0, D_SMALL), pl.ds(base, BPW)])

  return _sc_gather


def _mlp_body(e1, e2, e3t, w1a, w1b, w1s, b1, w2, b2, o_ref):
    h = jnp.dot(e1[...], w1a[...], preferred_element_type=jnp.float32)
    h += jnp.dot(e2[...], w1b[...], preferred_element_type=jnp.float32)
    h += lax.dot_general(e3t[...], w1s[...], (((0,), (0,)), ((), ())),
                         preferred_element_type=jnp.float32)
    h = jnp.maximum(h + b1[...], 0.0)
    o = jnp.dot(h, w2[...], preferred_element_type=jnp.float32)
    o_ref[...] = jnp.maximum(o + b2[...], 0.0)


BT = 2048


def _mlp(e1, e2, e3t, w1a, w1b, w1s, b1, w2, b2):
    full = lambda r, c: pl.BlockSpec((r, c), lambda i: (0, 0))
    return pl.pallas_call(
        _mlp_body,
        grid=(B // BT,),
        in_specs=[
            pl.BlockSpec((BT, 128), lambda i: (i, 0)),
            pl.BlockSpec((BT, 128), lambda i: (i, 0)),
            pl.BlockSpec((D_SMALL, BT), lambda i: (0, i)),
            full(128, 256), full(128, 256), full(D_SMALL, 256),
            full(1, 256), full(256, 64), full(1, 64),
        ],
        out_specs=pl.BlockSpec((BT, 64), lambda i: (i, 0)),
        out_shape=jax.ShapeDtypeStruct((B, 64), jnp.float32),
    )(e1, e2, e3t, w1a, w1b, w1s, b1.reshape(1, 256), w2, b2.reshape(1, 64))


def kernel(customer_id, article_id, product_type, colour_group, department,
           T_customer, T_article, T_product, T_colour, T_department,
           W1, b1, W2, b2):
    e1, e2, e3t = _make_sc_gather()(
        customer_id, article_id, product_type, colour_group, department,
        T_customer, T_article,
        T_product.reshape(-1), T_colour.reshape(-1), T_department.reshape(-1))
    return _mlp(e1, e2, e3t, W1[0:128], W1[128:256], W1[256:318], b1, W2, b2)


# speedup vs baseline: 8.2809x; 1.1465x over previous
import functools

import jax
import jax.numpy as jnp
from jax import lax
from jax.experimental import pallas as pl
from jax.experimental.pallas import tpu as pltpu
from jax.experimental.pallas import tpu_sc as plsc

B = 16384
NC, NS = 2, 16
NW = NC * NS
BPW = B // NW
CHUNK = 128
NCHUNK = BPW // CHUNK
L = 16
NGRP = BPW // L

D_PROD, D_COL, D_DEPT = 21, 17, 24
D_SMALL = D_PROD + D_COL + D_DEPT
V_PROD, V_COL, V_DEPT = 133, 51, 301


@functools.lru_cache(maxsize=1)
def _make_sc_gather():
  mesh = plsc.VectorSubcoreMesh(core_axis_name="c", subcore_axis_name="s",
                                num_cores=NC, num_subcores=NS)

  @functools.partial(
      pl.kernel,
      out_type=(jax.ShapeDtypeStruct((B, 256), jnp.float32),
                jax.ShapeDtypeStruct((D_SMALL, B), jnp.float32)),
      mesh=mesh,
      compiler_params=pltpu.CompilerParams(needs_layout_passes=False),
      scratch_types=(
          [pltpu.VMEM((BPW,), jnp.int32) for _ in range(5)]
          + [pltpu.VMEM((CHUNK, 128), jnp.float32) for _ in range(4)]
          + [pltpu.VMEM((V_PROD * D_PROD,), jnp.float32),
             pltpu.VMEM((V_COL * D_COL,), jnp.float32),
             pltpu.VMEM((V_DEPT * D_DEPT,), jnp.float32),
             pltpu.VMEM((D_SMALL, BPW), jnp.float32)]
          + [pltpu.SemaphoreType.DMA for _ in range(6)]
      ),
  )
  def _sc_gather(cid, aid, pid, gid, did, tc, ta, tpf, tgf, tdf,
                 e12, e3t,
                 ic, ia, ip, ig, idp, rc0, rc1, ra0, ra1,
                 tpv, tgv, tdv, rst,
                 sem_s, sem_w3, gsem0, gsem1, wsem0, wsem1):
      w = lax.axis_index("s") * NC + lax.axis_index("c")
      base = w * BPW
      ds_i = [
          pltpu.async_copy(cid.at[pl.ds(base, BPW)], ic, sem_s),
          pltpu.async_copy(aid.at[pl.ds(base, BPW)], ia, sem_s),
          pltpu.async_copy(pid.at[pl.ds(base, BPW)], ip, sem_s),
          pltpu.async_copy(gid.at[pl.ds(base, BPW)], ig, sem_s),
          pltpu.async_copy(did.at[pl.ds(base, BPW)], idp, sem_s),
      ]
      ds_t = [
          pltpu.async_copy(tpf, tpv, sem_s),
          pltpu.async_copy(tgf, tgv, sem_s),
          pltpu.async_copy(tdf, tdv, sem_s),
      ]
      for d in ds_i:
          d.wait()

      cbuf = (rc0, rc1)
      abuf = (ra0, ra1)
      gsem = (gsem0, gsem1)
      wsem = (wsem0, wsem1)

      def fire(c):
          off = c * CHUNK
          s = gsem[c % 2]
          return (
              pltpu.async_copy(tc.at[ic.at[pl.ds(off, CHUNK)]],
                               cbuf[c % 2], s),
              pltpu.async_copy(ta.at[ia.at[pl.ds(off, CHUNK)]],
                               abuf[c % 2], s),
          )

      gd = {0: fire(0), 1: fire(1)}

      for d in ds_t:
          d.wait()

      def group_body(g, carry):
          r0 = g * L
          pb = ip[pl.ds(r0, L)] * D_PROD
          gb = ig[pl.ds(r0, L)] * D_COL
          db = idp[pl.ds(r0, L)] * D_DEPT
          for d in range(D_PROD):
              rst[d, pl.ds(r0, L)] = plsc.load_gather(tpv, [pb + d])
          for d in range(D_COL):
              rst[D_PROD + d, pl.ds(r0, L)] = plsc.load_gather(tgv, [gb + d])
          for d in range(D_DEPT):
              rst[D_PROD + D_COL + d, pl.ds(r0, L)] = (
                  plsc.load_gather(tdv, [db + d]))
          return carry

      lax.fori_loop(0, NGRP, group_body, 0)
      w3 = pltpu.async_copy(rst, e3t.at[pl.ds(0, D_SMALL), pl.ds(base, BPW)],
                            sem_w3)

      wd = {}
      for c in range(NCHUNK):
          gd[c][0].wait()
          gd[c][1].wait()
          r0 = base + c * CHUNK
          s = wsem[c % 2]
          wd[c] = (
              pltpu.async_copy(cbuf[c % 2], e12.at[pl.ds(r0, CHUNK),
                                                   pl.ds(0, 128)], s),
              pltpu.async_copy(abuf[c % 2], e12.at[pl.ds(r0, CHUNK),
                                                   pl.ds(128, 128)], s),
          )
          if c + 2 < NCHUNK:
              wd[c][0].wait()
              wd[c][1].wait()
              gd[c + 2] = fire(c + 2)
      for c in range(max(0, NCHUNK - 2), NCHUNK):
          wd[c][0].wait()
          wd[c][1].wait()
      w3.wait()

  return _sc_gather


def _mlp_body(e12, e3t, w1ab, w1s, b1, w2, b2, o_ref):
    h = jnp.dot(e12[...], w1ab[...], preferred_element_type=jnp.float32)
    h += lax.dot_general(e3t[...], w1s[...], (((0,), (0,)), ((), ())),
                         preferred_element_type=jnp.float32)
    h = jnp.maximum(h + b1[...], 0.0)
    o = jnp.dot(h, w2[...], preferred_element_type=jnp.float32)
    o_ref[...] = jnp.maximum(o + b2[...], 0.0)


BT = 4096


def _mlp(e12, e3t, w1ab, w1s, b1, w2, b2):
    full = lambda r, c: pl.BlockSpec((r, c), lambda i: (0, 0))
    return pl.pallas_call(
        _mlp_body,
        grid=(B // BT,),
        in_specs=[
            pl.BlockSpec((BT, 256), lambda i: (i, 0)),
            pl.BlockSpec((D_SMALL, BT), lambda i: (0, i)),
            full(256, 256), full(D_SMALL, 256),
            full(1, 256), full(256, 64), full(1, 64),
        ],
        out_specs=pl.BlockSpec((BT, 64), lambda i: (i, 0)),
        out_shape=jax.ShapeDtypeStruct((B, 64), jnp.float32),
    )(e12, e3t, w1ab, w1s, b1.reshape(1, 256), w2, b2.reshape(1, 64))


def kernel(customer_id, article_id, product_type, colour_group, department,
           T_customer, T_article, T_product, T_colour, T_department,
           W1, b1, W2, b2):
    e12, e3t = _make_sc_gather()(
        customer_id, article_id, product_type, colour_group, department,
        T_customer, T_article,
        T_product.reshape(-1), T_colour.reshape(-1), T_department.reshape(-1))
    return _mlp(e12, e3t, W1[0:256], W1[256:318], b1, W2, b2)


# speedup vs baseline: 8.4144x; 1.0161x over previous
import functools

import jax
import jax.numpy as jnp
from jax import lax
from jax.experimental import pallas as pl
from jax.experimental.pallas import tpu as pltpu
from jax.experimental.pallas import tpu_sc as plsc

B = 16384
NC, NS = 2, 16
NW = NC * NS
BPW = B // NW
CHUNK = 64
NCHUNK = BPW // CHUNK
DEPTH = 3
L = 16
NGRP = BPW // L

D_PROD, D_COL, D_DEPT = 21, 17, 24
D_SMALL = D_PROD + D_COL + D_DEPT
V_PROD, V_COL, V_DEPT = 133, 51, 301
OFF_P = 0
OFF_G = V_PROD * D_PROD
OFF_D = OFF_G + V_COL * D_COL
TSM = OFF_D + V_DEPT * D_DEPT


@functools.lru_cache(maxsize=1)
def _make_sc_gather():
  mesh = plsc.VectorSubcoreMesh(core_axis_name="c", subcore_axis_name="s",
                                num_cores=NC, num_subcores=NS)

  @functools.partial(
      pl.kernel,
      out_type=(jax.ShapeDtypeStruct((B, 256), jnp.float32),
                jax.ShapeDtypeStruct((D_SMALL, B), jnp.float32)),
      mesh=mesh,
      compiler_params=pltpu.CompilerParams(needs_layout_passes=False),
      scratch_types=(
          [pltpu.VMEM((BPW,), jnp.int32) for _ in range(5)]
          + [pltpu.VMEM((CHUNK, 128), jnp.float32) for _ in range(2 * DEPTH)]
          + [pltpu.VMEM((TSM,), jnp.float32),
             pltpu.VMEM((D_SMALL, BPW), jnp.float32)]
          + [pltpu.SemaphoreType.DMA for _ in range(2 + 2 * DEPTH)]
      ),
  )
  def _sc_gather(cid, aid, pid, gid, did, tc, ta, tsm,
                 e12, e3t,
                 ic, ia, ip, ig, idp, rc0, rc1, rc2, ra0, ra1, ra2,
                 tsv, rst,
                 sem_s, sem_w3, gsem0, gsem1, gsem2, wsem0, wsem1, wsem2):
      w = lax.axis_index("s") * NC + lax.axis_index("c")
      base = w * BPW
      ds_i = [
          pltpu.async_copy(cid.at[pl.ds(base, BPW)], ic, sem_s),
          pltpu.async_copy(aid.at[pl.ds(base, BPW)], ia, sem_s),
          pltpu.async_copy(pid.at[pl.ds(base, BPW)], ip, sem_s),
          pltpu.async_copy(gid.at[pl.ds(base, BPW)], ig, sem_s),
          pltpu.async_copy(did.at[pl.ds(base, BPW)], idp, sem_s),
      ]
      dt = pltpu.async_copy(tsm, tsv, sem_s)
      for d in ds_i:
          d.wait()

      cbuf = (rc0, rc1, rc2)
      abuf = (ra0, ra1, ra2)
      gsem = (gsem0, gsem1, gsem2)
      wsem = (wsem0, wsem1, wsem2)

      def fire(c):
          off = c * CHUNK
          s = gsem[c % DEPTH]
          return (
              pltpu.async_copy(tc.at[ic.at[pl.ds(off, CHUNK)]],
                               cbuf[c % DEPTH], s),
              pltpu.async_copy(ta.at[ia.at[pl.ds(off, CHUNK)]],
                               abuf[c % DEPTH], s),
          )

      gd = {c: fire(c) for c in range(DEPTH)}

      dt.wait()

      def group_body(g, carry):
          r0 = g * L
          pb = ip[pl.ds(r0, L)] * D_PROD + OFF_P
          gb = ig[pl.ds(r0, L)] * D_COL + OFF_G
          db = idp[pl.ds(r0, L)] * D_DEPT + OFF_D
          for d in range(D_PROD):
              rst[d, pl.ds(r0, L)] = plsc.load_gather(tsv, [pb + d])
          for d in range(D_COL):
              rst[D_PROD + d, pl.ds(r0, L)] = plsc.load_gather(tsv, [gb + d])
          for d in range(D_DEPT):
              rst[D_PROD + D_COL + d, pl.ds(r0, L)] = (
                  plsc.load_gather(tsv, [db + d]))
          return carry

      lax.fori_loop(0, NGRP, group_body, 0)
      w3 = pltpu.async_copy(rst, e3t.at[pl.ds(0, D_SMALL), pl.ds(base, BPW)],
                            sem_w3)

      wd = {}
      for c in range(NCHUNK):
          gd[c][0].wait()
          gd[c][1].wait()
          r0 = base + c * CHUNK
          s = wsem[c % DEPTH]
          wd[c] = (
              pltpu.async_copy(cbuf[c % DEPTH], e12.at[pl.ds(r0, CHUNK),
                                                       pl.ds(0, 128)], s),
              pltpu.async_copy(abuf[c % DEPTH], e12.at[pl.ds(r0, CHUNK),
                                                       pl.ds(128, 128)], s),
          )
          if c + DEPTH < NCHUNK:
              wd[c][0].wait()
              wd[c][1].wait()
              gd[c + DEPTH] = fire(c + DEPTH)
      for c in range(max(0, NCHUNK - DEPTH), NCHUNK):
          wd[c][0].wait()
          wd[c][1].wait()
      w3.wait()

  return _sc_gather


def _mlp_body(e12, e3t, w1, b1, w2, b2, o_ref):
    h = jnp.dot(e12[...], w1[0:256], preferred_element_type=jnp.float32)
    h += lax.dot_general(e3t[...], w1[256:318], (((0,), (0,)), ((), ())),
                         preferred_element_type=jnp.float32)
    h = jnp.maximum(h + b1[...], 0.0)
    o = jnp.dot(h, w2[...], preferred_element_type=jnp.float32)
    o_ref[...] = jnp.maximum(o + b2[...], 0.0)


BT = 4096


def _mlp(e12, e3t, w1, b1, w2, b2):
    full = lambda r, c: pl.BlockSpec((r, c), lambda i: (0, 0))
    return pl.pallas_call(
        _mlp_body,
        grid=(B // BT,),
        in_specs=[
            pl.BlockSpec((BT, 256), lambda i: (i, 0)),
            pl.BlockSpec((D_SMALL, BT), lambda i: (0, i)),
            full(318, 256), full(1, 256), full(256, 64), full(1, 64),
        ],
        out_specs=pl.BlockSpec((BT, 64), lambda i: (i, 0)),
        out_shape=jax.ShapeDtypeStruct((B, 64), jnp.float32),
    )(e12, e3t, w1, b1.reshape(1, 256), w2, b2.reshape(1, 64))


def kernel(customer_id, article_id, product_type, colour_group, department,
           T_customer, T_article, T_product, T_colour, T_department,
           W1, b1, W2, b2):
    tsm = jnp.concatenate([T_product.reshape(-1), T_colour.reshape(-1),
                           T_department.reshape(-1)])
    e12, e3t = _make_sc_gather()(
        customer_id, article_id, product_type, colour_group, department,
        T_customer, T_article, tsm)
    return _mlp(e12, e3t, W1, b1, W2, b2)


# speedup vs baseline: 9.4924x; 1.1281x over previous
import functools

import jax
import jax.numpy as jnp
from jax import lax
from jax.experimental import pallas as pl
from jax.experimental.pallas import tpu as pltpu
from jax.experimental.pallas import tpu_sc as plsc

B = 16384
NC, NS = 2, 16
NW = NC * NS
BPW = B // NW
CHUNK = 64
NCHUNK = BPW // CHUNK
DEPTH = 3
L = 16
NGRP = BPW // L

D_PROD, D_COL, D_DEPT = 21, 17, 24
D_SMALL = D_PROD + D_COL + D_DEPT
V_PROD, V_COL, V_DEPT = 133, 51, 301
OFF_P = 0
OFF_G = V_PROD * D_PROD
OFF_D = OFF_G + V_COL * D_COL
TSM = OFF_D + V_DEPT * D_DEPT


@functools.lru_cache(maxsize=1)
def _make_sc_gather():
  mesh = plsc.VectorSubcoreMesh(core_axis_name="c", subcore_axis_name="s",
                                num_cores=NC, num_subcores=NS)

  @functools.partial(
      pl.kernel,
      out_type=(jax.ShapeDtypeStruct((B, 256), jnp.float32),
                jax.ShapeDtypeStruct((D_SMALL, B), jnp.float32)),
      mesh=mesh,
      compiler_params=pltpu.CompilerParams(needs_layout_passes=False),
      scratch_types=(
          [pltpu.VMEM((BPW,), jnp.int32) for _ in range(5)]
          + [pltpu.VMEM((CHUNK, 128), jnp.float32) for _ in range(2 * DEPTH)]
          + [pltpu.VMEM((TSM,), jnp.float32),
             pltpu.VMEM((D_SMALL, BPW), jnp.float32)]
          + [pltpu.SemaphoreType.DMA for _ in range(2 + 2 * DEPTH)]
      ),
  )
  def _sc_gather(cid, aid, pid, gid, did, tc, ta, tsm,
                 e12, e3t,
                 ic, ia, ip, ig, idp, rc0, rc1, rc2, ra0, ra1, ra2,
                 tsv, rst,
                 sem_s, sem_w3, gsem0, gsem1, gsem2, wsem0, wsem1, wsem2):
      w = lax.axis_index("s") * NC + lax.axis_index("c")
      base = w * BPW
      d_ic = pltpu.async_copy(cid.at[pl.ds(base, BPW)], ic, sem_s)
      d_ia = pltpu.async_copy(aid.at[pl.ds(base, BPW)], ia, sem_s)
      ds_i = [
          pltpu.async_copy(pid.at[pl.ds(base, BPW)], ip, sem_s),
          pltpu.async_copy(gid.at[pl.ds(base, BPW)], ig, sem_s),
          pltpu.async_copy(did.at[pl.ds(base, BPW)], idp, sem_s),
      ]
      dt = pltpu.async_copy(tsm, tsv, sem_s)
      d_ic.wait()
      d_ia.wait()

      cbuf = (rc0, rc1, rc2)
      abuf = (ra0, ra1, ra2)
      gsem = (gsem0, gsem1, gsem2)
      wsem = (wsem0, wsem1, wsem2)

      def fire(c):
          off = c * CHUNK
          s = gsem[c % DEPTH]
          return (
              pltpu.async_copy(tc.at[ic.at[pl.ds(off, CHUNK)]],
                               cbuf[c % DEPTH], s),
              pltpu.async_copy(ta.at[ia.at[pl.ds(off, CHUNK)]],
                               abuf[c % DEPTH], s),
          )

      gd = {c: fire(c) for c in range(DEPTH)}

      for d in ds_i:
          d.wait()
      dt.wait()

      ILP = 8

      def group_body(g, carry):
          r0 = g * L
          pb = ip[pl.ds(r0, L)] * D_PROD + OFF_P
          gb = ig[pl.ds(r0, L)] * D_COL + OFF_G
          db = idp[pl.ds(r0, L)] * D_DEPT + OFF_D
          cols = ([(d, pb + d) for d in range(D_PROD)]
                  + [(D_PROD + d, gb + d) for d in range(D_COL)]
                  + [(D_PROD + D_COL + d, db + d) for d in range(D_DEPT)])
          for i in range(0, D_SMALL, ILP):
              batch = cols[i:i + ILP]
              vals = [plsc.load_gather(tsv, [idx]) for _, idx in batch]
              for (row, _), v in zip(batch, vals):
                  rst[row, pl.ds(r0, L)] = v
          return carry

      lax.fori_loop(0, NGRP, group_body, 0)
      w3 = pltpu.async_copy(rst, e3t.at[pl.ds(0, D_SMALL), pl.ds(base, BPW)],
                            sem_w3)

      wd = {}
      for c in range(NCHUNK):
          gd[c][0].wait()
          gd[c][1].wait()
          r0 = base + c * CHUNK
          s = wsem[c % DEPTH]
          wd[c] = (
              pltpu.async_copy(cbuf[c % DEPTH], e12.at[pl.ds(r0, CHUNK),
                                                       pl.ds(0, 128)], s),
              pltpu.async_copy(abuf[c % DEPTH], e12.at[pl.ds(r0, CHUNK),
                                                       pl.ds(128, 128)], s),
          )
          if c + DEPTH < NCHUNK:
              wd[c][0].wait()
              wd[c][1].wait()
              gd[c + DEPTH] = fire(c + DEPTH)
      for c in range(max(0, NCHUNK - DEPTH), NCHUNK):
          wd[c][0].wait()
          wd[c][1].wait()
      w3.wait()

  return _sc_gather


def _mlp_body(e12, e3t, w1, b1, w2, b2, o_ref):
    h = jnp.dot(e12[...], w1[0:256], preferred_element_type=jnp.float32)
    h += lax.dot_general(e3t[...], w1[256:318], (((0,), (0,)), ((), ())),
                         preferred_element_type=jnp.float32)
    h = jnp.maximum(h + b1[...], 0.0)
    o = jnp.dot(h, w2[...], preferred_element_type=jnp.float32)
    o_ref[...] = jnp.maximum(o + b2[...], 0.0)


BT = 4096


def _mlp(e12, e3t, w1, b1, w2, b2):
    full = lambda r, c: pl.BlockSpec((r, c), lambda i: (0, 0))
    return pl.pallas_call(
        _mlp_body,
        grid=(B // BT,),
        in_specs=[
            pl.BlockSpec((BT, 256), lambda i: (i, 0)),
            pl.BlockSpec((D_SMALL, BT), lambda i: (0, i)),
            full(318, 256), full(1, 256), full(256, 64), full(1, 64),
        ],
        out_specs=pl.BlockSpec((BT, 64), lambda i: (i, 0)),
        out_shape=jax.ShapeDtypeStruct((B, 64), jnp.float32),
    )(e12, e3t, w1, b1.reshape(1, 256), w2, b2.reshape(1, 64))


def kernel(customer_id, article_id, product_type, colour_group, department,
           T_customer, T_article, T_product, T_colour, T_department,
           W1, b1, W2, b2):
    tsm = jnp.concatenate([T_product.reshape(-1), T_colour.reshape(-1),
                           T_department.reshape(-1)])
    e12, e3t = _make_sc_gather()(
        customer_id, article_id, product_type, colour_group, department,
        T_customer, T_article, tsm)
    return _mlp(e12, e3t, W1, b1, W2, b2)
